# baseline (device time: 15148 ns/iter reference)
import jax
import jax.numpy as jnp
from jax import lax
from jax.experimental import pallas as pl
from jax.experimental.pallas import tpu as pltpu

N_SUB = 4
HALF = 256
SUB = HALF // N_SUB


def kernel(partial, resid, gamma):
    _, m, d = partial.shape
    gamma2 = gamma.reshape(1, d)

    def body(p_ref, r_hbm, g_ref, o_ref, send_half, recv_buf, r_vmem,
             y_send_sems, y_recv_sems, f_send_sems, f_recv_sems, copy_sem):
        my_x = lax.axis_index("x")
        my_y = lax.axis_index("y")
        my_z = lax.axis_index("z")
        y_nbr = (my_x, 1 - my_y, my_z)
        x_nbr = (1 - my_x, my_y, my_z)
        z_nbr = (my_x, my_y, 1 - my_z)

        par = (my_x + my_z) % 2
        h0 = par * HALF
        c0 = (1 - par) * HALF

        bsem = pltpu.get_barrier_semaphore()
        for nbr in (y_nbr, x_nbr, z_nbr):
            pl.semaphore_signal(
                bsem, inc=1, device_id=nbr, device_id_type=pl.DeviceIdType.MESH
            )

        cp = pltpu.make_async_copy(r_hbm, r_vmem, copy_sem)
        cp.start()

        send_half[...] = p_ref[0, pl.ds(h0, HALF), :].astype(jnp.bfloat16)
        pl.semaphore_wait(bsem, 3)

        y_rdmas = []
        for j in range(N_SUB):
            rdma = pltpu.make_async_remote_copy(
                src_ref=send_half.at[pl.ds(j * SUB, SUB)],
                dst_ref=recv_buf.at[pl.ds(h0 + j * SUB, SUB)],
                send_sem=y_send_sems.at[j],
                recv_sem=y_recv_sems.at[j],
                device_id=y_nbr,
                device_id_type=pl.DeviceIdType.MESH,
            )
            rdma.start()
            y_rdmas.append(rdma)

        cp.wait()

        def norm_store(row0):
            blk = pl.ds(row0, SUB)
            y = (p_ref[0, blk, :] + recv_buf[blk, :].astype(jnp.float32)
                 + r_vmem[blk, :])
            ms = jnp.mean(y * y, axis=-1, keepdims=True)
            o_ref[blk, :] = (y * lax.rsqrt(ms + 1e-6)
                             * g_ref[...]).astype(jnp.bfloat16)

        f_rdmas = []
        for j in range(N_SUB):
            y_rdmas[j].wait()
            tgt = x_nbr if j < 2 else z_nbr
            fwd = pltpu.make_async_remote_copy(
                src_ref=recv_buf.at[pl.ds(h0 + j * SUB, SUB)],
                dst_ref=recv_buf.at[pl.ds(h0 + j * SUB, SUB)],
                send_sem=f_send_sems.at[j],
                recv_sem=f_recv_sems.at[j],
                device_id=tgt,
                device_id_type=pl.DeviceIdType.MESH,
            )
            fwd.start()
            f_rdmas.append(fwd)
            norm_store(h0 + j * SUB)

        for j in range(N_SUB):
            src = x_nbr if j < 2 else z_nbr
            rcv = pltpu.make_async_remote_copy(
                src_ref=recv_buf.at[pl.ds(c0 + j * SUB, SUB)],
                dst_ref=recv_buf.at[pl.ds(c0 + j * SUB, SUB)],
                send_sem=f_send_sems.at[j],
                recv_sem=f_recv_sems.at[j],
                device_id=src,
                device_id_type=pl.DeviceIdType.MESH,
            )
            rcv.wait_recv()
            norm_store(c0 + j * SUB)

        for j in range(N_SUB):
            f_rdmas[j].wait_send()

    return pl.pallas_call(
        body,
        out_shape=jax.ShapeDtypeStruct((m, d), jnp.bfloat16),
        in_specs=[
            pl.BlockSpec(memory_space=pltpu.VMEM),
            pl.BlockSpec(memory_space=pl.ANY),
            pl.BlockSpec(memory_space=pltpu.VMEM),
        ],
        out_specs=pl.BlockSpec(memory_space=pltpu.VMEM),
        scratch_shapes=[
            pltpu.VMEM((HALF, d), jnp.bfloat16),
            pltpu.VMEM((m, d), jnp.bfloat16),
            pltpu.VMEM((m, d), jnp.float32),
            pltpu.SemaphoreType.DMA((N_SUB,)),
            pltpu.SemaphoreType.DMA((N_SUB,)),
            pltpu.SemaphoreType.DMA((N_SUB,)),
            pltpu.SemaphoreType.DMA((N_SUB,)),
            pltpu.SemaphoreType.DMA,
        ],
        compiler_params=pltpu.CompilerParams(collective_id=0),
    )(partial, resid, gamma2)


# device time: 14214 ns/iter; 1.0657x vs baseline; 1.0657x over previous
import jax
import jax.numpy as jnp
from jax import lax
from jax.experimental import pallas as pl
from jax.experimental.pallas import tpu as pltpu

N_SUB = 4
HALF = 256
SUB = HALF // N_SUB


def kernel(partial, resid, gamma):
    _, m, d = partial.shape
    gamma2 = gamma.reshape(1, d)

    def body(p_ref, r_hbm, g_ref, o_ref, send_half, recv_buf, r_vmem,
             y_send_sems, y_recv_sems, f_send_sems, f_recv_sems, copy_sem):
        my_x = lax.axis_index("x")
        my_y = lax.axis_index("y")
        my_z = lax.axis_index("z")
        y_nbr = (my_x, 1 - my_y, my_z)
        x_nbr = (1 - my_x, my_y, my_z)

        par = my_x
        h0 = par * HALF
        c0 = (1 - par) * HALF

        bsem = pltpu.get_barrier_semaphore()
        for nbr in (y_nbr, x_nbr):
            pl.semaphore_signal(
                bsem, inc=1, device_id=nbr, device_id_type=pl.DeviceIdType.MESH
            )

        cp = pltpu.make_async_copy(r_hbm, r_vmem, copy_sem)
        cp.start()

        send_half[...] = p_ref[0, pl.ds(h0, HALF), :].astype(jnp.bfloat16)
        pl.semaphore_wait(bsem, 2)

        y_rdmas = []
        for j in range(N_SUB):
            rdma = pltpu.make_async_remote_copy(
                src_ref=send_half.at[pl.ds(j * SUB, SUB)],
                dst_ref=recv_buf.at[pl.ds(h0 + j * SUB, SUB)],
                send_sem=y_send_sems.at[j],
                recv_sem=y_recv_sems.at[j],
                device_id=y_nbr,
                device_id_type=pl.DeviceIdType.MESH,
            )
            rdma.start()
            y_rdmas.append(rdma)

        cp.wait()

        def norm_store(row0):
            blk = pl.ds(row0, SUB)
            y = (p_ref[0, blk, :] + recv_buf[blk, :].astype(jnp.float32)
                 + r_vmem[blk, :])
            ms = jnp.mean(y * y, axis=-1, keepdims=True)
            o_ref[blk, :] = (y * lax.rsqrt(ms + 1e-6)
                             * g_ref[...]).astype(jnp.bfloat16)

        f_rdmas = []
        for j in range(N_SUB):
            y_rdmas[j].wait()
            tgt = x_nbr
            fwd = pltpu.make_async_remote_copy(
                src_ref=recv_buf.at[pl.ds(h0 + j * SUB, SUB)],
                dst_ref=recv_buf.at[pl.ds(h0 + j * SUB, SUB)],
                send_sem=f_send_sems.at[j],
                recv_sem=f_recv_sems.at[j],
                device_id=tgt,
                device_id_type=pl.DeviceIdType.MESH,
            )
            fwd.start()
            f_rdmas.append(fwd)
            norm_store(h0 + j * SUB)

        for j in range(N_SUB):
            src = x_nbr
            rcv = pltpu.make_async_remote_copy(
                src_ref=recv_buf.at[pl.ds(c0 + j * SUB, SUB)],
                dst_ref=recv_buf.at[pl.ds(c0 + j * SUB, SUB)],
                send_sem=f_send_sems.at[j],
                recv_sem=f_recv_sems.at[j],
                device_id=src,
                device_id_type=pl.DeviceIdType.MESH,
            )
            rcv.wait_recv()
            norm_store(c0 + j * SUB)

        for j in range(N_SUB):
            f_rdmas[j].wait_send()

    return pl.pallas_call(
        body,
        out_shape=jax.ShapeDtypeStruct((m, d), jnp.bfloat16),
        in_specs=[
            pl.BlockSpec(memory_space=pltpu.VMEM),
            pl.BlockSpec(memory_space=pl.ANY),
            pl.BlockSpec(memory_space=pltpu.VMEM),
        ],
        out_specs=pl.BlockSpec(memory_space=pltpu.VMEM),
        scratch_shapes=[
            pltpu.VMEM((HALF, d), jnp.bfloat16),
            pltpu.VMEM((m, d), jnp.bfloat16),
            pltpu.VMEM((m, d), jnp.float32),
            pltpu.SemaphoreType.DMA((N_SUB,)),
            pltpu.SemaphoreType.DMA((N_SUB,)),
            pltpu.SemaphoreType.DMA((N_SUB,)),
            pltpu.SemaphoreType.DMA((N_SUB,)),
            pltpu.SemaphoreType.DMA,
        ],
        compiler_params=pltpu.CompilerParams(collective_id=0),
    )(partial, resid, gamma2)


# device time: 13898 ns/iter; 1.0899x vs baseline; 1.0227x over previous
import jax
import jax.numpy as jnp
from jax import lax
from jax.experimental import pallas as pl
from jax.experimental.pallas import tpu as pltpu

N_CHUNKS = 8


def kernel(partial, resid, gamma):
    _, m, d = partial.shape
    gamma2 = gamma.reshape(1, d)
    rows = m // N_CHUNKS

    def body(p_ref, r_hbm, g_ref, o_ref, send_buf, recv_buf, r_vmem,
             send_sems, recv_sems, copy_sem):
        my_x = lax.axis_index("x")
        my_y = lax.axis_index("y")
        my_z = lax.axis_index("z")
        nbr = (my_x, 1 - my_y, my_z)

        bsem = pltpu.get_barrier_semaphore()
        pl.semaphore_signal(
            bsem, inc=1, device_id=nbr, device_id_type=pl.DeviceIdType.MESH
        )

        cp = pltpu.make_async_copy(r_hbm, r_vmem, copy_sem)
        cp.start()

        send_buf[...] = p_ref[0, :, :].astype(jnp.bfloat16)
        pl.semaphore_wait(bsem, 1)

        rdmas = []
        for k in range(N_CHUNKS):
            blk = pl.ds(k * rows, rows)
            rdma = pltpu.make_async_remote_copy(
                src_ref=send_buf.at[blk],
                dst_ref=recv_buf.at[blk],
                send_sem=send_sems.at[k],
                recv_sem=recv_sems.at[k],
                device_id=nbr,
                device_id_type=pl.DeviceIdType.MESH,
            )
            rdma.start()
            rdmas.append(rdma)

        cp.wait()
        for k in range(N_CHUNKS):
            rdmas[k].wait()
            blk = pl.ds(k * rows, rows)
            y = (p_ref[0, blk, :] + recv_buf[blk, :].astype(jnp.float32)
                 + r_vmem[blk, :])
            ms = jnp.mean(y * y, axis=-1, keepdims=True)
            o_ref[blk, :] = (y * lax.rsqrt(ms + 1e-6)
                             * g_ref[...]).astype(jnp.bfloat16)

    return pl.pallas_call(
        body,
        out_shape=jax.ShapeDtypeStruct((m, d), jnp.bfloat16),
        in_specs=[
            pl.BlockSpec(memory_space=pltpu.VMEM),
            pl.BlockSpec(memory_space=pl.ANY),
            pl.BlockSpec(memory_space=pltpu.VMEM),
        ],
        out_specs=pl.BlockSpec(memory_space=pltpu.VMEM),
        scratch_shapes=[
            pltpu.VMEM((m, d), jnp.bfloat16),
            pltpu.VMEM((m, d), jnp.bfloat16),
            pltpu.VMEM((m, d), jnp.float32),
            pltpu.SemaphoreType.DMA((N_CHUNKS,)),
            pltpu.SemaphoreType.DMA((N_CHUNKS,)),
            pltpu.SemaphoreType.DMA,
        ],
        compiler_params=pltpu.CompilerParams(collective_id=0),
    )(partial, resid, gamma2)
